# baseline (speedup 1.0000x reference)
import functools

import jax
import jax.numpy as jnp
from jax import lax
from jax.experimental import pallas as pl
from jax.experimental.pallas import tpu as pltpu
from jax.experimental.pallas import tpu_sc as plsc

VOCAB = 32
D = 64
B = 16384
H = 200
NW = 32
BW = B // NW
CH_H = 8
NCH = H // CH_H
NC16 = BW // 16


def _sc_embed(seq_t, table):
    mesh = plsc.VectorSubcoreMesh(core_axis_name="c", subcore_axis_name="s")

    @functools.partial(
        pl.kernel,
        out_type=jax.ShapeDtypeStruct((H * D, B), jnp.float32),
        mesh=mesh,
        compiler_params=pltpu.CompilerParams(needs_layout_passes=False),
        scratch_types=[
            pltpu.VMEM((VOCAB, D), jnp.float32),
            pltpu.VMEM((VOCAB * D,), jnp.float32),
            pltpu.VMEM((2, CH_H, BW), jnp.int32),
            pltpu.VMEM((2, D, BW), jnp.float32),
            pltpu.SemaphoreType.DMA,
            pltpu.SemaphoreType.DMA,
        ],
    )
    def k(seq_hbm, table_hbm, out_hbm, table_v, lut_v, sidx_v, obuf_v,
          wsem, isem):
        cid = lax.axis_index("c")
        sid = lax.axis_index("s")
        wid = sid * 2 + cid
        b0 = wid * BW
        lanes = lax.iota(jnp.int32, 16)

        pltpu.sync_copy(table_hbm, table_v)
        for v in range(VOCAB):
            for c in range(4):
                x = table_v[v, pl.ds(c * 16, 16)]
                dix = (lanes + c * 16) * VOCAB + v
                plsc.store_scatter(lut_v, [dix], x)

        def idx_copy(ch, buf):
            return pltpu.make_async_copy(
                seq_hbm.at[pl.ds(ch * CH_H, CH_H), pl.ds(b0, BW)],
                sidx_v.at[buf], isem)

        def write_copy(h, buf):
            return pltpu.make_async_copy(
                obuf_v.at[buf], out_hbm.at[pl.ds(h * D, D), pl.ds(b0, BW)],
                wsem)

        idx_copy(0, 0).start()

        @pl.loop(0, NCH)
        def _chunk(ch):
            sb = lax.rem(ch, 2)
            idx_copy(ch, sb).wait()

            @pl.when(ch + 1 < NCH)
            def _():
                idx_copy(ch + 1, 1 - sb).start()

            for hl in range(CH_H):
                h = ch * CH_H + hl
                buf = hl % 2
                if hl >= 2:
                    write_copy(0, buf).wait()
                else:
                    @pl.when(ch > 0)
                    def _():
                        write_copy(0, buf).wait()

                idxs = [sidx_v[sb, hl, pl.ds(c * 16, 16)] for c in range(NC16)]

                @plsc.parallel_loop(0, D, unroll=8)
                def _row(d):
                    lut_d = lut_v.at[pl.ds(d * VOCAB, VOCAB)]
                    for c in range(NC16):
                        g = plsc.load_gather(lut_d, [idxs[c]])
                        obuf_v[buf, d, pl.ds(c * 16, 16)] = g

                write_copy(h, buf).start()

        for _ in range(2):
            write_copy(0, 0).wait()

    return k(seq_t, table)


def kernel(seq, table):
    out2d = _sc_embed(jnp.swapaxes(seq, 0, 1), table)
    return out2d.reshape(H, D, B).transpose(2, 0, 1)


# speedup vs baseline: 1.2429x; 1.2429x over previous
import functools

import jax
import jax.numpy as jnp
from jax import lax
from jax.experimental import pallas as pl
from jax.experimental.pallas import tpu as pltpu
from jax.experimental.pallas import tpu_sc as plsc

VOCAB = 32
D = 64
B = 16384
H = 200
NW = 32
BW = B // NW
CH_H = 8
NCH = H // CH_H
NC16 = BW // 16


def _sc_embed(seq_t, table):
    mesh = plsc.VectorSubcoreMesh(core_axis_name="c", subcore_axis_name="s")

    @functools.partial(
        pl.kernel,
        out_type=jax.ShapeDtypeStruct((H * D, B), jnp.float32),
        mesh=mesh,
        compiler_params=pltpu.CompilerParams(needs_layout_passes=False),
        scratch_types=[
            pltpu.VMEM((VOCAB, D), jnp.float32),
            pltpu.VMEM((VOCAB * D,), jnp.float32),
            pltpu.VMEM((2, CH_H, BW), jnp.int32),
            pltpu.VMEM((2, D, BW), jnp.float32),
            pltpu.SemaphoreType.DMA,
            pltpu.SemaphoreType.DMA,
        ],
    )
    def k(seq_hbm, table_hbm, out_hbm, table_v, lut_v, sidx_v, obuf_v,
          wsem, isem):
        cid = lax.axis_index("c")
        sid = lax.axis_index("s")
        wid = sid * 2 + cid
        b0 = wid * BW
        lanes = lax.iota(jnp.int32, 16)

        pltpu.sync_copy(table_hbm, table_v)
        for v in range(VOCAB):
            for c in range(4):
                x = table_v[v, pl.ds(c * 16, 16)]
                dix = (lanes + c * 16) * VOCAB + v
                plsc.store_scatter(lut_v, [dix], x)

        def idx_copy(ch, buf):
            return pltpu.make_async_copy(
                seq_hbm.at[pl.ds(ch * CH_H, CH_H), pl.ds(b0, BW)],
                sidx_v.at[buf], isem)

        def write_copy(h, buf):
            return pltpu.make_async_copy(
                obuf_v.at[buf], out_hbm.at[pl.ds(h * D, D), pl.ds(b0, BW)],
                wsem)

        idx_copy(0, 0).start()

        @pl.loop(0, NCH)
        def _chunk(ch):
            sb = lax.rem(ch, 2)
            idx_copy(ch, sb).wait()

            @pl.when(ch + 1 < NCH)
            def _():
                idx_copy(ch + 1, 1 - sb).start()

            for hl in range(CH_H):
                h = ch * CH_H + hl
                buf = hl % 2
                if hl >= 2:
                    write_copy(0, buf).wait()
                else:
                    @pl.when(ch > 0)
                    def _():
                        write_copy(0, buf).wait()

                idxs = [sidx_v[sb, hl, pl.ds(c * 16, 16)] for c in range(NC16)]

                @plsc.parallel_loop(0, D, unroll=2)
                def _row(d):
                    lut_d = lut_v.at[pl.ds(d * VOCAB, VOCAB)]
                    for c in range(NC16):
                        g = plsc.load_gather(lut_d, [idxs[c]])
                        obuf_v[buf, d, pl.ds(c * 16, 16)] = g

                write_copy(h, buf).start()

        for _ in range(2):
            write_copy(0, 0).wait()

    return k(seq_t, table)


def kernel(seq, table):
    out2d = _sc_embed(jnp.swapaxes(seq, 0, 1), table)
    return out2d.reshape(H, D, B).transpose(2, 0, 1)


# speedup vs baseline: 1.3952x; 1.1226x over previous
import functools

import jax
import jax.numpy as jnp
from jax import lax
from jax.experimental import pallas as pl
from jax.experimental.pallas import tpu as pltpu
from jax.experimental.pallas import tpu_sc as plsc

VOCAB = 32
D = 64
B = 16384
H = 200
NW = 32
BW = B // NW
CH_H = 8
NCH = H // CH_H
NC16 = BW // 16


def _sc_embed(seq_t, table):
    mesh = plsc.VectorSubcoreMesh(core_axis_name="c", subcore_axis_name="s")

    @functools.partial(
        pl.kernel,
        out_type=jax.ShapeDtypeStruct((H * D, B), jnp.float32),
        mesh=mesh,
        compiler_params=pltpu.CompilerParams(needs_layout_passes=False),
        scratch_types=[
            pltpu.VMEM((VOCAB, D), jnp.float32),
            pltpu.VMEM((VOCAB * D,), jnp.float32),
            pltpu.VMEM((2, CH_H, BW), jnp.int32),
            pltpu.VMEM((2, D, BW), jnp.float32),
            pltpu.SemaphoreType.DMA,
            pltpu.SemaphoreType.DMA,
        ],
    )
    def k(seq_hbm, table_hbm, out_hbm, table_v, lut_v, sidx_v, obuf_v,
          wsem, isem):
        cid = lax.axis_index("c")
        sid = lax.axis_index("s")
        wid = sid * 2 + cid
        b0 = wid * BW
        lanes = lax.iota(jnp.int32, 16)

        pltpu.sync_copy(table_hbm, table_v)
        for v in range(VOCAB):
            for c in range(4):
                x = table_v[v, pl.ds(c * 16, 16)]
                dix = (lanes + c * 16) * VOCAB + v
                plsc.store_scatter(lut_v, [dix], x)

        def idx_copy(ch, buf):
            return pltpu.make_async_copy(
                seq_hbm.at[pl.ds(ch * CH_H, CH_H), pl.ds(b0, BW)],
                sidx_v.at[buf], isem)

        def write_copy(h, buf):
            return pltpu.make_async_copy(
                obuf_v.at[buf], out_hbm.at[pl.ds(h * D, D), pl.ds(b0, BW)],
                wsem)

        idx_copy(0, 0).start()

        @pl.loop(0, NCH)
        def _chunk(ch):
            sb = lax.rem(ch, 2)
            idx_copy(ch, sb).wait()

            @pl.when(ch + 1 < NCH)
            def _():
                idx_copy(ch + 1, 1 - sb).start()

            for hl in range(CH_H):
                h = ch * CH_H + hl
                buf = hl % 2
                if hl >= 2:
                    write_copy(0, buf).wait()
                else:
                    @pl.when(ch > 0)
                    def _():
                        write_copy(0, buf).wait()

                idxs = [sidx_v[sb, hl, pl.ds(c * 16, 16)] for c in range(NC16)]

                @plsc.parallel_loop(0, D)
                def _row(d):
                    lut_d = lut_v.at[pl.ds(d * VOCAB, VOCAB)]
                    for c in range(NC16):
                        g = plsc.load_gather(lut_d, [idxs[c]])
                        obuf_v[buf, d, pl.ds(c * 16, 16)] = g

                write_copy(h, buf).start()

        for _ in range(2):
            write_copy(0, 0).wait()

    return k(seq_t, table)


def kernel(seq, table):
    out2d = _sc_embed(jnp.swapaxes(seq, 0, 1), table)
    return out2d.reshape(H, D, B).transpose(2, 0, 1)


# speedup vs baseline: 1.4352x; 1.0287x over previous
import functools

import jax
import jax.numpy as jnp
from jax import lax
from jax.experimental import pallas as pl
from jax.experimental.pallas import tpu as pltpu
from jax.experimental.pallas import tpu_sc as plsc

VOCAB = 32
D = 64
B = 16384
H = 200
NW = 32
BW = B // NW
CH_H = 8
NCH = H // CH_H
NC16 = BW // 16


def _sc_embed(seq_t, table):
    mesh = plsc.VectorSubcoreMesh(core_axis_name="c", subcore_axis_name="s")

    @functools.partial(
        pl.kernel,
        out_type=jax.ShapeDtypeStruct((H * D, B), jnp.float32),
        mesh=mesh,
        compiler_params=pltpu.CompilerParams(needs_layout_passes=False),
        scratch_types=[
            pltpu.VMEM((VOCAB, D), jnp.float32),
            pltpu.VMEM((VOCAB * D,), jnp.float32),
            pltpu.VMEM((2, CH_H, BW), jnp.int32),
            pltpu.VMEM((3, D, BW), jnp.float32),
            pltpu.SemaphoreType.DMA,
            pltpu.SemaphoreType.DMA,
        ],
    )
    def k(seq_hbm, table_hbm, out_hbm, table_v, lut_v, sidx_v, obuf_v,
          wsem, isem):
        cid = lax.axis_index("c")
        sid = lax.axis_index("s")
        wid = sid * 2 + cid
        b0 = wid * BW
        lanes = lax.iota(jnp.int32, 16)

        pltpu.sync_copy(table_hbm, table_v)
        for v in range(VOCAB):
            for c in range(4):
                x = table_v[v, pl.ds(c * 16, 16)]
                dix = (lanes + c * 16) * VOCAB + v
                plsc.store_scatter(lut_v, [dix], x)

        def idx_copy(ch, buf):
            return pltpu.make_async_copy(
                seq_hbm.at[pl.ds(ch * CH_H, CH_H), pl.ds(b0, BW)],
                sidx_v.at[buf], isem)

        def write_copy(h, buf):
            return pltpu.make_async_copy(
                obuf_v.at[buf], out_hbm.at[pl.ds(h * D, D), pl.ds(b0, BW)],
                wsem)

        idx_copy(0, 0).start()

        @pl.loop(0, NCH)
        def _chunk(ch):
            sb = lax.rem(ch, 2)
            idx_copy(ch, sb).wait()

            @pl.when(ch + 1 < NCH)
            def _():
                idx_copy(ch + 1, 1 - sb).start()

            for hl in range(CH_H):
                h = ch * CH_H + hl
                buf = lax.rem(ch * CH_H + hl, 3)
                if hl >= 3:
                    write_copy(0, 0).wait()
                else:
                    @pl.when(ch > 0)
                    def _():
                        write_copy(0, 0).wait()

                idxs = [sidx_v[sb, hl, pl.ds(c * 16, 16)] for c in range(NC16)]

                @plsc.parallel_loop(0, D)
                def _row(d):
                    lut_d = lut_v.at[pl.ds(d * VOCAB, VOCAB)]
                    for c in range(NC16):
                        g = plsc.load_gather(lut_d, [idxs[c]])
                        obuf_v[buf, d, pl.ds(c * 16, 16)] = g

                write_copy(h, buf).start()

        for _ in range(3):
            write_copy(0, 0).wait()

    return k(seq_t, table)


def kernel(seq, table):
    out2d = _sc_embed(jnp.swapaxes(seq, 0, 1), table)
    return out2d.reshape(H, D, B).transpose(2, 0, 1)
